# Initial kernel scaffold; baseline (speedup 1.0000x reference)
import jax
import jax.numpy as jnp
from jax.experimental import pallas as pl


def kernel(x, edge_index, W1, al1, ar1, b1, W2, al2, ar2, b2, W3, al3, ar3, b3):
    raise NotImplementedError("write your pallas kernel here")


# speedup vs baseline: 10.0835x; 10.0835x over previous
import functools

import jax
import jax.numpy as jnp
from jax import lax
from jax.experimental import pallas as pl
from jax.experimental.pallas import tpu as pltpu
from jax.experimental.pallas import tpu_sc as plsc

N = 10000
E = 160000
F = 256
HALF = 128
NT = 16
EPT = E // NT
NV = EPT // 16
NP = 10240
STRIPE = NP // NT
LANES = 16
NR = NP // HALF
KC = 80
NCH = EPT // KC
RD = 16


def _tc_body(xa_ref, xb_ref, wa_ref, wb_ref, a_ref, h0_ref, h1_ref, ea_ref):
    h = jnp.dot(xa_ref[...], wa_ref[...], preferred_element_type=jnp.float32)
    h = h + jnp.dot(xb_ref[...], wb_ref[...], preferred_element_type=jnp.float32)
    h0_ref[...] = h[:, :HALF]
    h1_ref[...] = h[:, HALF:]
    ea_ref[...] = jnp.dot(h, a_ref[...], preferred_element_type=jnp.float32)


def _tc_project(xa, xb, wa, wb, amat):
    mb = 1024
    return pl.pallas_call(
        _tc_body,
        grid=(NP // mb,),
        in_specs=[
            pl.BlockSpec((mb, HALF), lambda i: (i, 0)),
            pl.BlockSpec((mb, HALF), lambda i: (i, 0)),
            pl.BlockSpec((HALF, F), lambda i: (0, 0)),
            pl.BlockSpec((HALF, F), lambda i: (0, 0)),
            pl.BlockSpec((F, HALF), lambda i: (0, 0)),
        ],
        out_specs=[
            pl.BlockSpec((mb, HALF), lambda i: (i, 0)),
            pl.BlockSpec((mb, HALF), lambda i: (i, 0)),
            pl.BlockSpec((mb, HALF), lambda i: (i, 0)),
        ],
        out_shape=[
            jax.ShapeDtypeStruct((NP, HALF), jnp.float32),
            jax.ShapeDtypeStruct((NP, HALF), jnp.float32),
            jax.ShapeDtypeStruct((NP, HALF), jnp.float32),
        ],
    )(xa, xb, wa, wb, amat)


_SC_MESH = plsc.VectorSubcoreMesh(core_axis_name="c", subcore_axis_name="s")


@functools.partial(
    pl.kernel,
    out_type=[
        jax.ShapeDtypeStruct((E,), jnp.float32),
        jax.ShapeDtypeStruct((NR, HALF), jnp.float32),
    ],
    mesh=_SC_MESH,
    compiler_params=pltpu.CompilerParams(needs_layout_passes=False),
    scratch_types=[
        pltpu.VMEM((NR, HALF), jnp.float32),
        pltpu.VMEM((EPT,), jnp.float32),
        pltpu.VMEM((EPT,), jnp.int32),
        pltpu.VMEM((EPT,), jnp.int32),
        pltpu.VMEM((NR,), jnp.int32),
        pltpu.VMEM_SHARED((NR, HALF), jnp.float32),
        pltpu.SemaphoreType.DMA,
    ],
)
def _sc_stats(el_hbm, er_hbm, src_hbm, dst_hbm, ee_hbm, den_hbm,
              work_v, alp_v, src_v, dst_v, idxden, den_sh, sem):
    c = lax.axis_index("c")
    s = lax.axis_index("s")
    ebase = s * EPT
    lane = lax.iota(jnp.int32, LANES)
    c127 = jnp.full((LANES,), 127, jnp.int32)

    pltpu.sync_copy(el_hbm, work_v)
    pltpu.sync_copy(src_hbm.at[pl.ds(ebase, EPT)], src_v)
    pltpu.sync_copy(dst_hbm.at[pl.ds(ebase, EPT)], dst_v)
    for i in range(NR // LANES):
        idxden[pl.ds(i * LANES, LANES)] = lane + (i * LANES)

    def _mx(i, m):
        return jnp.maximum(m, work_v[i >> 3, pl.ds((i & 7) * LANES, LANES)])
    mvec = lax.fori_loop(0, NP // LANES, _mx,
                         jnp.full((LANES,), -jnp.inf, jnp.float32))

    def _lane_gather(v, idx):
        dn = lax.GatherDimensionNumbers(
            offset_dims=(), collapsed_slice_dims=(0,), start_index_map=(0,))
        return lax.gather(v, idx[:, None], dn, (1,),
                          mode=lax.GatherScatterMode.PROMISE_IN_BOUNDS)

    for sh in (1, 2, 4, 8):
        mvec = jnp.maximum(mvec, _lane_gather(mvec, lane ^ sh))
    maxel = mvec

    def _l1a(j, _):
        si = src_v[pl.ds(j * LANES, LANES)]
        g1 = plsc.load_gather(
            work_v, [lax.shift_right_logical(si, 7), si & c127])
        alp_v[pl.ds(j * LANES, LANES)] = g1
        return 0
    lax.fori_loop(0, NV, _l1a, 0)

    pltpu.sync_copy(er_hbm, work_v)

    def _l1b(j, _):
        sl = pl.ds(j * LANES, LANES)
        di = dst_v[sl]
        g2 = plsc.load_gather(
            work_v, [lax.shift_right_logical(di, 7), di & c127])
        t = alp_v[sl] + g2
        e = jnp.where(t >= 0, t, 0.2 * t)
        mt = maxel + g2
        md = jnp.where(mt >= 0, mt, 0.2 * mt)
        alp_v[sl] = jnp.exp(e - md)
        return 0
    lax.fori_loop(0, NV, _l1b, 0)

    def _zw(r, _):
        for v in range(HALF // LANES):
            work_v[r, pl.ds(v * LANES, LANES)] = jnp.zeros((LANES,),
                                                           jnp.float32)
        return 0
    lax.fori_loop(0, NR, _zw, 0)

    @pl.when(s < 10)
    def _():
        pltpu.sync_copy(work_v.at[pl.ds(0, 8)], den_sh.at[pl.ds(s * 8, 8)])
    plsc.subcore_barrier()

    def _l2(j, _):
        sl = pl.ds(j * LANES, LANES)
        di = dst_v[sl]
        plsc.addupdate_scatter(
            work_v, [lax.shift_right_logical(di, 7), di & c127], alp_v[sl])
        return 0
    lax.fori_loop(0, NV, _l2, 0)

    pltpu.async_copy(work_v, den_sh.at[idxden], sem, add=True).wait()
    plsc.subcore_barrier()

    @pl.when(c == 0)
    def _():
        pltpu.sync_copy(alp_v, ee_hbm.at[pl.ds(ebase, EPT)])

        @pl.when(s < 10)
        def _():
            pltpu.sync_copy(den_sh.at[pl.ds(s * 8, 8)],
                            den_hbm.at[pl.ds(s * 8, 8)])


@functools.partial(
    pl.kernel,
    out_type=[
        jax.ShapeDtypeStruct((NP, HALF), jnp.float32),
        jax.ShapeDtypeStruct((NP, HALF), jnp.float32),
    ],
    mesh=_SC_MESH,
    compiler_params=pltpu.CompilerParams(needs_layout_passes=False),
    scratch_types=[
        pltpu.VMEM((NR, HALF), jnp.float32),
        pltpu.VMEM((KC,), jnp.int32),
        pltpu.VMEM((1, KC), jnp.int32),
        pltpu.VMEM((KC,), jnp.float32),
        pltpu.VMEM((HALF,), jnp.float32),
        pltpu.VMEM((HALF,), jnp.float32),
        pltpu.VMEM_SHARED((NP, HALF), jnp.float32),
        pltpu.SemaphoreType.DMA,
        pltpu.SemaphoreType.DMA,
    ],
)
def _sc_agg(h0_hbm, h1_hbm, ee_hbm, den_hbm, src_hbm, dst_hbm, b0_hbm, b1_hbm,
            z0_hbm, z1_hbm,
            work_v, srcc_v, idx2, eec_v, denb_v, bias_v, acc_sh, gsem, ssem):
    c = lax.axis_index("c")
    s = lax.axis_index("s")
    ebase = s * EPT

    def _zw(r, _):
        for v in range(HALF // LANES):
            work_v[r, pl.ds(v * LANES, LANES)] = jnp.zeros((LANES,),
                                                           jnp.float32)
        return 0
    lax.fori_loop(0, NR, _zw, 0)
    for q in range(STRIPE // NR):
        pltpu.sync_copy(work_v, acc_sh.at[pl.ds(s * STRIPE + q * NR, NR)])
    plsc.subcore_barrier()

    def _phase_c(h_ref):
        def _chunk(j, _):
            eb = ebase + j * KC
            pltpu.sync_copy(src_hbm.at[pl.ds(eb, KC)], srcc_v)
            pltpu.sync_copy(dst_hbm.at[pl.ds(eb, KC)], idx2.at[0])
            pltpu.sync_copy(ee_hbm.at[pl.ds(eb, KC)], eec_v)
            pltpu.async_copy(h_ref.at[srcc_v], work_v, gsem).wait()

            def _scale(g, _2):
                av = eec_v[pl.ds(g * LANES, LANES)]
                for k in range(LANES):
                    r = g * LANES + k
                    ab = jnp.broadcast_to(av[k], (LANES,))
                    for v in range(HALF // LANES):
                        sl = pl.ds(v * LANES, LANES)
                        work_v[r, sl] = work_v[r, sl] * ab
                return 0
            lax.fori_loop(0, KC // LANES, _scale, 0)
            pltpu.async_copy(work_v, acc_sh.at[idx2.at[0]], ssem,
                             add=True).wait()
            return 0
        lax.fori_loop(0, NCH, _chunk, 0)

    @pl.when(c == 0)
    def _():
        _phase_c(h0_hbm)

    @pl.when(c == 1)
    def _():
        _phase_c(h1_hbm)

    plsc.subcore_barrier()

    def _phase_d(z_ref, b_ref):
        pltpu.sync_copy(b_ref, bias_v)
        for p in range(STRIPE // HALF):
            pltpu.sync_copy(den_hbm.at[s * (STRIPE // HALF) + p], denb_v)

            def _dchunk(q, _):
                r0 = s * STRIPE + p * HALF + q * RD
                pltpu.sync_copy(acc_sh.at[pl.ds(r0, RD)],
                                work_v.at[pl.ds(0, RD)])
                dd = denb_v[pl.ds(q * RD, LANES)]
                dd = jnp.where(dd > 0.0, dd, 1.0)
                for r in range(RD):
                    db = jnp.broadcast_to(dd[r], (LANES,))
                    for v in range(HALF // LANES):
                        sl = pl.ds(v * LANES, LANES)
                        val = work_v[r, sl] / db + bias_v[sl]
                        work_v[r, sl] = jnp.maximum(val, 0.0)
                pltpu.sync_copy(work_v.at[pl.ds(0, RD)],
                                z_ref.at[pl.ds(r0, RD)])
                return 0
            lax.fori_loop(0, HALF // RD, _dchunk, 0)

    @pl.when(c == 0)
    def _():
        _phase_d(z0_hbm, b0_hbm)

    @pl.when(c == 1)
    def _():
        _phase_d(z1_hbm, b1_hbm)


def _layer(xa, xb, src, dst, W, al, ar, b):
    wa = W[:HALF]
    wb = W[HALF:]
    amat = jnp.zeros((F, HALF), jnp.float32)
    amat = amat.at[:, 0].set(al).at[:, 1].set(ar)
    h0, h1, ea = _tc_project(xa, xb, wa, wb, amat)
    el = ea[:, 0].reshape(NR, HALF)
    er = ea[:, 1].reshape(NR, HALF)
    ee, den = _sc_stats(el, er, src, dst)
    z0, z1 = _sc_agg(h0, h1, ee, den, src, dst, b[:HALF], b[HALF:])
    return z0, z1


@jax.jit
def kernel(x, edge_index, W1, al1, ar1, b1, W2, al2, ar2, b2, W3, al3, ar3, b3):
    src = edge_index[0]
    dst = edge_index[1]
    xp = jnp.pad(x, ((0, NP - N), (0, 0)))
    xa = xp[:, :HALF]
    xb = xp[:, HALF:]
    for (W, al, ar, b) in ((W1, al1, ar1, b1), (W2, al2, ar2, b2),
                           (W3, al3, ar3, b3)):
        xa, xb = _layer(xa, xb, src, dst, W, al, ar, b)
    return jnp.concatenate([xa, xb], axis=1)[:N]


# speedup vs baseline: 12.9467x; 1.2840x over previous
import functools

import jax
import jax.numpy as jnp
from jax import lax
from jax.experimental import pallas as pl
from jax.experimental.pallas import tpu as pltpu
from jax.experimental.pallas import tpu_sc as plsc

N = 10000
E = 160000
F = 256
HALF = 128
NT = 16
EPT = E // NT
NV = EPT // 16
NP = 10240
STRIPE = NP // NT
LANES = 16
NR = NP // HALF
KC = 80
NCH = EPT // KC
RD = 16


def _tc_body(xa_ref, xb_ref, wa_ref, wb_ref, a_ref, h0_ref, h1_ref, ea_ref):
    h = jnp.dot(xa_ref[...], wa_ref[...], preferred_element_type=jnp.float32)
    h = h + jnp.dot(xb_ref[...], wb_ref[...], preferred_element_type=jnp.float32)
    h0_ref[...] = h[:, :HALF]
    h1_ref[...] = h[:, HALF:]
    ea_ref[...] = jnp.dot(h, a_ref[...], preferred_element_type=jnp.float32)


def _tc_project(xa, xb, wa, wb, amat):
    mb = 1024
    return pl.pallas_call(
        _tc_body,
        grid=(NP // mb,),
        in_specs=[
            pl.BlockSpec((mb, HALF), lambda i: (i, 0)),
            pl.BlockSpec((mb, HALF), lambda i: (i, 0)),
            pl.BlockSpec((HALF, F), lambda i: (0, 0)),
            pl.BlockSpec((HALF, F), lambda i: (0, 0)),
            pl.BlockSpec((F, HALF), lambda i: (0, 0)),
        ],
        out_specs=[
            pl.BlockSpec((mb, HALF), lambda i: (i, 0)),
            pl.BlockSpec((mb, HALF), lambda i: (i, 0)),
            pl.BlockSpec((mb, HALF), lambda i: (i, 0)),
        ],
        out_shape=[
            jax.ShapeDtypeStruct((NP, HALF), jnp.float32),
            jax.ShapeDtypeStruct((NP, HALF), jnp.float32),
            jax.ShapeDtypeStruct((NP, HALF), jnp.float32),
        ],
    )(xa, xb, wa, wb, amat)


_SC_MESH = plsc.VectorSubcoreMesh(core_axis_name="c", subcore_axis_name="s")


@functools.partial(
    pl.kernel,
    out_type=[
        jax.ShapeDtypeStruct((E,), jnp.float32),
        jax.ShapeDtypeStruct((NR, HALF), jnp.float32),
    ],
    mesh=_SC_MESH,
    compiler_params=pltpu.CompilerParams(needs_layout_passes=False),
    scratch_types=[
        pltpu.VMEM((NR, HALF), jnp.float32),
        pltpu.VMEM((EPT,), jnp.float32),
        pltpu.VMEM((EPT,), jnp.int32),
        pltpu.VMEM((EPT,), jnp.int32),
        pltpu.VMEM((NR,), jnp.int32),
        pltpu.VMEM_SHARED((NR, HALF), jnp.float32),
        pltpu.SemaphoreType.DMA,
    ],
)
def _sc_stats(el_hbm, er_hbm, src_hbm, dst_hbm, ee_hbm, den_hbm,
              work_v, alp_v, src_v, dst_v, idxden, den_sh, sem):
    c = lax.axis_index("c")
    s = lax.axis_index("s")
    ebase = s * EPT
    lane = lax.iota(jnp.int32, LANES)
    c127 = jnp.full((LANES,), 127, jnp.int32)

    pltpu.sync_copy(el_hbm, work_v)
    pltpu.sync_copy(src_hbm.at[pl.ds(ebase, EPT)], src_v)
    pltpu.sync_copy(dst_hbm.at[pl.ds(ebase, EPT)], dst_v)
    for i in range(NR // LANES):
        idxden[pl.ds(i * LANES, LANES)] = lane + (i * LANES)

    def _mx(i, m):
        return jnp.maximum(m, work_v[i >> 3, pl.ds((i & 7) * LANES, LANES)])
    mvec = lax.fori_loop(0, NP // LANES, _mx,
                         jnp.full((LANES,), -jnp.inf, jnp.float32))

    def _lane_gather(v, idx):
        dn = lax.GatherDimensionNumbers(
            offset_dims=(), collapsed_slice_dims=(0,), start_index_map=(0,))
        return lax.gather(v, idx[:, None], dn, (1,),
                          mode=lax.GatherScatterMode.PROMISE_IN_BOUNDS)

    for sh in (1, 2, 4, 8):
        mvec = jnp.maximum(mvec, _lane_gather(mvec, lane ^ sh))
    maxel = mvec

    def _l1a(j, _):
        si = src_v[pl.ds(j * LANES, LANES)]
        g1 = plsc.load_gather(
            work_v, [lax.shift_right_logical(si, 7), si & c127])
        alp_v[pl.ds(j * LANES, LANES)] = g1
        return 0
    lax.fori_loop(0, NV, _l1a, 0)

    pltpu.sync_copy(er_hbm, work_v)

    def _l1b(j, _):
        sl = pl.ds(j * LANES, LANES)
        di = dst_v[sl]
        g2 = plsc.load_gather(
            work_v, [lax.shift_right_logical(di, 7), di & c127])
        t = alp_v[sl] + g2
        e = jnp.where(t >= 0, t, 0.2 * t)
        mt = maxel + g2
        md = jnp.where(mt >= 0, mt, 0.2 * mt)
        alp_v[sl] = jnp.exp(e - md)
        return 0
    lax.fori_loop(0, NV, _l1b, 0)

    def _zw(r, _):
        for v in range(HALF // LANES):
            work_v[r, pl.ds(v * LANES, LANES)] = jnp.zeros((LANES,),
                                                           jnp.float32)
        return 0
    lax.fori_loop(0, NR, _zw, 0)

    @pl.when(s < 10)
    def _():
        pltpu.sync_copy(work_v.at[pl.ds(0, 8)], den_sh.at[pl.ds(s * 8, 8)])
    plsc.subcore_barrier()

    def _l2(j, _):
        sl = pl.ds(j * LANES, LANES)
        di = dst_v[sl]
        plsc.addupdate_scatter(
            work_v, [lax.shift_right_logical(di, 7), di & c127], alp_v[sl])
        return 0
    lax.fori_loop(0, NV, _l2, 0)

    pltpu.async_copy(work_v, den_sh.at[idxden], sem, add=True).wait()
    plsc.subcore_barrier()

    @pl.when(c == 0)
    def _():
        pltpu.sync_copy(alp_v, ee_hbm.at[pl.ds(ebase, EPT)])

        @pl.when(s < 10)
        def _():
            pltpu.sync_copy(den_sh.at[pl.ds(s * 8, 8)],
                            den_hbm.at[pl.ds(s * 8, 8)])


@functools.partial(
    pl.kernel,
    out_type=[
        jax.ShapeDtypeStruct((NP, HALF), jnp.float32),
        jax.ShapeDtypeStruct((NP, HALF), jnp.float32),
    ],
    mesh=_SC_MESH,
    compiler_params=pltpu.CompilerParams(needs_layout_passes=False),
    scratch_types=[
        pltpu.VMEM((2, KC, HALF), jnp.float32),
        pltpu.VMEM((2, KC), jnp.int32),
        pltpu.VMEM((2, KC), jnp.int32),
        pltpu.VMEM((2, KC), jnp.float32),
        pltpu.VMEM((RD, HALF), jnp.float32),
        pltpu.VMEM((HALF,), jnp.float32),
        pltpu.VMEM((HALF,), jnp.float32),
        pltpu.VMEM_SHARED((NP, HALF), jnp.float32),
        pltpu.SemaphoreType.DMA,
        pltpu.SemaphoreType.DMA,
    ],
)
def _sc_agg(h0_hbm, h1_hbm, ee_hbm, den_hbm, src_hbm, dst_hbm, b0_hbm, b1_hbm,
            z0_hbm, z1_hbm,
            rows_v, srcc_v, idx2, eec_v, outw_v, denb_v, bias_v, acc_sh,
            gsem, ssem):
    c = lax.axis_index("c")
    s = lax.axis_index("s")
    ebase = s * EPT

    def _zr(r, _):
        for v in range(HALF // LANES):
            rows_v[0, r, pl.ds(v * LANES, LANES)] = jnp.zeros((LANES,),
                                                              jnp.float32)
        return 0
    lax.fori_loop(0, KC, _zr, 0)

    for q in range(STRIPE // NR):
        pltpu.sync_copy(rows_v.at[0], acc_sh.at[pl.ds(s * STRIPE + q * NR,
                                                      NR)])
    plsc.subcore_barrier()

    def _phase_c(h_ref):
        def _stage(ch, b):
            eb = ebase + ch * KC
            pltpu.sync_copy(src_hbm.at[pl.ds(eb, KC)], srcc_v.at[b])
            pltpu.sync_copy(dst_hbm.at[pl.ds(eb, KC)], idx2.at[b])
            pltpu.sync_copy(ee_hbm.at[pl.ds(eb, KC)], eec_v.at[b])
            pltpu.async_copy(h_ref.at[srcc_v.at[b]], rows_v.at[b], gsem)

        def _wait_gather(b):
            pltpu.make_async_copy(h_ref.at[srcc_v.at[b]], rows_v.at[b],
                                  gsem).wait()

        def _scale(b):
            def _sg(g, _2):
                av = eec_v[b, pl.ds(g * LANES, LANES)]
                for k in range(LANES):
                    r = g * LANES + k
                    ab = jnp.broadcast_to(av[k], (LANES,))
                    for v in range(HALF // LANES):
                        sl = pl.ds(v * LANES, LANES)
                        rows_v[b, r, sl] = rows_v[b, r, sl] * ab
                return 0
            lax.fori_loop(0, KC // LANES, _sg, 0)

        def _scatter(b):
            return pltpu.async_copy(rows_v.at[b], acc_sh.at[idx2.at[b]],
                                    ssem, add=True)

        def _pair(m, _):
            _stage(2 * m, 0)
            _stage(2 * m + 1, 1)
            _wait_gather(0)
            _scale(0)
            d0 = _scatter(0)
            _wait_gather(1)
            _scale(1)
            d1 = _scatter(1)
            d0.wait()
            d1.wait()
            return 0
        lax.fori_loop(0, NCH // 2, _pair, 0)

        _stage(NCH - 1, 0)
        _wait_gather(0)
        _scale(0)
        _scatter(0).wait()

    @pl.when(c == 0)
    def _():
        _phase_c(h0_hbm)

    @pl.when(c == 1)
    def _():
        _phase_c(h1_hbm)

    plsc.subcore_barrier()

    plsc.subcore_barrier()

    def _phase_d(z_ref, b_ref):
        pltpu.sync_copy(b_ref, bias_v)
        for p in range(STRIPE // HALF):
            pltpu.sync_copy(den_hbm.at[s * (STRIPE // HALF) + p], denb_v)

            def _dchunk(q, _):
                r0 = s * STRIPE + p * HALF + q * RD
                pltpu.sync_copy(acc_sh.at[pl.ds(r0, RD)], outw_v)
                dd = denb_v[pl.ds(q * RD, LANES)]
                dd = jnp.where(dd > 0.0, dd, 1.0)
                for r in range(RD):
                    db = jnp.broadcast_to(dd[r], (LANES,))
                    for v in range(HALF // LANES):
                        sl = pl.ds(v * LANES, LANES)
                        val = outw_v[r, sl] / db + bias_v[sl]
                        outw_v[r, sl] = jnp.maximum(val, 0.0)
                pltpu.sync_copy(outw_v, z_ref.at[pl.ds(r0, RD)])
                return 0
            lax.fori_loop(0, HALF // RD, _dchunk, 0)

    @pl.when(c == 0)
    def _():
        _phase_d(z0_hbm, b0_hbm)

    @pl.when(c == 1)
    def _():
        _phase_d(z1_hbm, b1_hbm)


def _layer(xa, xb, src, dst, W, al, ar, b):
    wa = W[:HALF]
    wb = W[HALF:]
    amat = jnp.zeros((F, HALF), jnp.float32)
    amat = amat.at[:, 0].set(al).at[:, 1].set(ar)
    h0, h1, ea = _tc_project(xa, xb, wa, wb, amat)
    el = ea[:, 0].reshape(NR, HALF)
    er = ea[:, 1].reshape(NR, HALF)
    ee, den = _sc_stats(el, er, src, dst)
    z0, z1 = _sc_agg(h0, h1, ee, den, src, dst, b[:HALF], b[HALF:])
    return z0, z1


@jax.jit
def kernel(x, edge_index, W1, al1, ar1, b1, W2, al2, ar2, b2, W3, al3, ar3, b3):
    src = edge_index[0]
    dst = edge_index[1]
    xp = jnp.pad(x, ((0, NP - N), (0, 0)))
    xa = xp[:, :HALF]
    xb = xp[:, HALF:]
    for (W, al, ar, b) in ((W1, al1, ar1, b1), (W2, al2, ar2, b2),
                           (W3, al3, ar3, b3)):
        xa, xb = _layer(xa, xb, src, dst, W, al, ar, b)
    return jnp.concatenate([xa, xb], axis=1)[:N]


# speedup vs baseline: 14.1208x; 1.0907x over previous
import functools

import jax
import jax.numpy as jnp
from jax import lax
from jax.experimental import pallas as pl
from jax.experimental.pallas import tpu as pltpu
from jax.experimental.pallas import tpu_sc as plsc

N = 10000
E = 160000
F = 256
HALF = 128
NT = 16
EPT = E // NT
NV = EPT // 16
NP = 10240
STRIPE = NP // NT
LANES = 16
NR = NP // HALF
KC = 80
NCH = EPT // KC
RD = 16


def _tc_body(xa_ref, xb_ref, wa_ref, wb_ref, a_ref, h0_ref, h1_ref, ea_ref):
    h = jnp.dot(xa_ref[...], wa_ref[...], preferred_element_type=jnp.float32)
    h = h + jnp.dot(xb_ref[...], wb_ref[...], preferred_element_type=jnp.float32)
    h0_ref[...] = h[:, :HALF]
    h1_ref[...] = h[:, HALF:]
    ea_ref[...] = jnp.dot(h, a_ref[...], preferred_element_type=jnp.float32)


def _tc_project(xa, xb, wa, wb, amat):
    mb = 1024
    return pl.pallas_call(
        _tc_body,
        grid=(NP // mb,),
        in_specs=[
            pl.BlockSpec((mb, HALF), lambda i: (i, 0)),
            pl.BlockSpec((mb, HALF), lambda i: (i, 0)),
            pl.BlockSpec((HALF, F), lambda i: (0, 0)),
            pl.BlockSpec((HALF, F), lambda i: (0, 0)),
            pl.BlockSpec((F, HALF), lambda i: (0, 0)),
        ],
        out_specs=[
            pl.BlockSpec((mb, HALF), lambda i: (i, 0)),
            pl.BlockSpec((mb, HALF), lambda i: (i, 0)),
            pl.BlockSpec((mb, HALF), lambda i: (i, 0)),
        ],
        out_shape=[
            jax.ShapeDtypeStruct((NP, HALF), jnp.float32),
            jax.ShapeDtypeStruct((NP, HALF), jnp.float32),
            jax.ShapeDtypeStruct((NP, HALF), jnp.float32),
        ],
    )(xa, xb, wa, wb, amat)


_SC_MESH = plsc.VectorSubcoreMesh(core_axis_name="c", subcore_axis_name="s")


@functools.partial(
    pl.kernel,
    out_type=[
        jax.ShapeDtypeStruct((E,), jnp.float32),
        jax.ShapeDtypeStruct((NR, HALF), jnp.float32),
    ],
    mesh=_SC_MESH,
    compiler_params=pltpu.CompilerParams(needs_layout_passes=False),
    scratch_types=[
        pltpu.VMEM((NR, HALF), jnp.float32),
        pltpu.VMEM((EPT,), jnp.float32),
        pltpu.VMEM((EPT,), jnp.int32),
        pltpu.VMEM((EPT,), jnp.int32),
        pltpu.VMEM((NR,), jnp.int32),
        pltpu.VMEM_SHARED((NR, HALF), jnp.float32),
        pltpu.SemaphoreType.DMA,
    ],
)
def _sc_stats(el_hbm, er_hbm, src_hbm, dst_hbm, ee_hbm, den_hbm,
              work_v, alp_v, src_v, dst_v, idxden, den_sh, sem):
    c = lax.axis_index("c")
    s = lax.axis_index("s")
    ebase = s * EPT
    lane = lax.iota(jnp.int32, LANES)
    c127 = jnp.full((LANES,), 127, jnp.int32)

    pltpu.sync_copy(el_hbm, work_v)
    pltpu.sync_copy(src_hbm.at[pl.ds(ebase, EPT)], src_v)
    pltpu.sync_copy(dst_hbm.at[pl.ds(ebase, EPT)], dst_v)
    for i in range(NR // LANES):
        idxden[pl.ds(i * LANES, LANES)] = lane + (i * LANES)

    def _mx(i, m):
        return jnp.maximum(m, work_v[i >> 3, pl.ds((i & 7) * LANES, LANES)])
    mvec = lax.fori_loop(0, NP // LANES, _mx,
                         jnp.full((LANES,), -jnp.inf, jnp.float32))

    def _lane_gather(v, idx):
        dn = lax.GatherDimensionNumbers(
            offset_dims=(), collapsed_slice_dims=(0,), start_index_map=(0,))
        return lax.gather(v, idx[:, None], dn, (1,),
                          mode=lax.GatherScatterMode.PROMISE_IN_BOUNDS)

    for sh in (1, 2, 4, 8):
        mvec = jnp.maximum(mvec, _lane_gather(mvec, lane ^ sh))
    maxel = mvec

    def _l1a(j, _):
        si = src_v[pl.ds(j * LANES, LANES)]
        g1 = plsc.load_gather(
            work_v, [lax.shift_right_logical(si, 7), si & c127])
        alp_v[pl.ds(j * LANES, LANES)] = g1
        return 0
    lax.fori_loop(0, NV, _l1a, 0)

    pltpu.sync_copy(er_hbm, work_v)

    def _l1b(j, _):
        sl = pl.ds(j * LANES, LANES)
        di = dst_v[sl]
        g2 = plsc.load_gather(
            work_v, [lax.shift_right_logical(di, 7), di & c127])
        t = alp_v[sl] + g2
        e = jnp.where(t >= 0, t, 0.2 * t)
        mt = maxel + g2
        md = jnp.where(mt >= 0, mt, 0.2 * mt)
        alp_v[sl] = jnp.exp(e - md)
        return 0
    lax.fori_loop(0, NV, _l1b, 0)

    def _zw(r, _):
        for v in range(HALF // LANES):
            work_v[r, pl.ds(v * LANES, LANES)] = jnp.zeros((LANES,),
                                                           jnp.float32)
        return 0
    lax.fori_loop(0, NR, _zw, 0)

    @pl.when(s < 10)
    def _():
        pltpu.sync_copy(work_v.at[pl.ds(0, 8)], den_sh.at[pl.ds(s * 8, 8)])
    plsc.subcore_barrier()

    def _l2(j, _):
        sl = pl.ds(j * LANES, LANES)
        di = dst_v[sl]
        plsc.addupdate_scatter(
            work_v, [lax.shift_right_logical(di, 7), di & c127], alp_v[sl])
        return 0
    lax.fori_loop(0, NV, _l2, 0)

    pltpu.async_copy(work_v, den_sh.at[idxden], sem, add=True).wait()
    plsc.subcore_barrier()

    @pl.when(c == 0)
    def _():
        pltpu.sync_copy(alp_v, ee_hbm.at[pl.ds(ebase, EPT)])

        @pl.when(s < 10)
        def _():
            pltpu.sync_copy(den_sh.at[pl.ds(s * 8, 8)],
                            den_hbm.at[pl.ds(s * 8, 8)])


@functools.partial(
    pl.kernel,
    out_type=[
        jax.ShapeDtypeStruct((NP, HALF), jnp.float32),
        jax.ShapeDtypeStruct((NP, HALF), jnp.float32),
    ],
    mesh=_SC_MESH,
    compiler_params=pltpu.CompilerParams(needs_layout_passes=False),
    scratch_types=[
        pltpu.VMEM((3, KC, HALF), jnp.float32),
        pltpu.VMEM((3, KC), jnp.int32),
        pltpu.VMEM((3, KC), jnp.int32),
        pltpu.VMEM((3, KC), jnp.float32),
        pltpu.VMEM((RD, HALF), jnp.float32),
        pltpu.VMEM((HALF,), jnp.float32),
        pltpu.VMEM((HALF,), jnp.float32),
        pltpu.VMEM_SHARED((NP, HALF), jnp.float32),
        pltpu.SemaphoreType.DMA,
        pltpu.SemaphoreType.DMA,
    ],
)
def _sc_agg(h0_hbm, h1_hbm, ee_hbm, den_hbm, src_hbm, dst_hbm, b0_hbm, b1_hbm,
            z0_hbm, z1_hbm,
            rows_v, srcc_v, idx2, eec_v, outw_v, denb_v, bias_v, acc_sh,
            gsem, ssem):
    c = lax.axis_index("c")
    s = lax.axis_index("s")
    ebase = s * EPT

    def _zr(r, _):
        for v in range(HALF // LANES):
            rows_v[0, r, pl.ds(v * LANES, LANES)] = jnp.zeros((LANES,),
                                                              jnp.float32)
        return 0
    lax.fori_loop(0, KC, _zr, 0)

    for q in range(STRIPE // NR):
        pltpu.sync_copy(rows_v.at[0], acc_sh.at[pl.ds(s * STRIPE + q * NR,
                                                      NR)])
    plsc.subcore_barrier()

    def _phase_c(h_ref):
        def _stage(ch, b):
            eb = ebase + ch * KC
            pltpu.sync_copy(src_hbm.at[pl.ds(eb, KC)], srcc_v.at[b])
            pltpu.sync_copy(dst_hbm.at[pl.ds(eb, KC)], idx2.at[b])
            pltpu.sync_copy(ee_hbm.at[pl.ds(eb, KC)], eec_v.at[b])
            return pltpu.async_copy(h_ref.at[srcc_v.at[b]], rows_v.at[b],
                                    gsem)

        def _scale(b):
            def _sg(g, _2):
                av = eec_v[b, pl.ds(g * LANES, LANES)]
                for k in range(LANES):
                    r = g * LANES + k
                    ab = jnp.broadcast_to(av[k], (LANES,))
                    for v in range(HALF // LANES):
                        sl = pl.ds(v * LANES, LANES)
                        rows_v[b, r, sl] = rows_v[b, r, sl] * ab
                return 0
            lax.fori_loop(0, KC // LANES, _sg, 0)

        def _scatter(b):
            return pltpu.async_copy(rows_v.at[b], acc_sh.at[idx2.at[b]],
                                    ssem, add=True)

        def _group(m, _):
            ch = 3 * m
            g = [_stage(ch + b, b) for b in range(3)]
            d = []
            for b in range(3):
                g[b].wait()
                _scale(b)
                d.append(_scatter(b))
            for b in range(3):
                d[b].wait()
            return 0
        lax.fori_loop(0, NCH // 3, _group, 0)

        ch0 = (NCH // 3) * 3
        g = [_stage(ch0 + b, b) for b in range(NCH % 3)]
        d = []
        for b in range(NCH % 3):
            g[b].wait()
            _scale(b)
            d.append(_scatter(b))
        for b in range(NCH % 3):
            d[b].wait()

    @pl.when(c == 0)
    def _():
        _phase_c(h0_hbm)

    @pl.when(c == 1)
    def _():
        _phase_c(h1_hbm)

    plsc.subcore_barrier()

    plsc.subcore_barrier()

    def _phase_d(z_ref, b_ref):
        pltpu.sync_copy(b_ref, bias_v)
        for p in range(STRIPE // HALF):
            pltpu.sync_copy(den_hbm.at[s * (STRIPE // HALF) + p], denb_v)

            def _dchunk(q, _):
                r0 = s * STRIPE + p * HALF + q * RD
                pltpu.sync_copy(acc_sh.at[pl.ds(r0, RD)], outw_v)
                dd = denb_v[pl.ds(q * RD, LANES)]
                dd = jnp.where(dd > 0.0, dd, 1.0)
                for r in range(RD):
                    db = jnp.broadcast_to(dd[r], (LANES,))
                    for v in range(HALF // LANES):
                        sl = pl.ds(v * LANES, LANES)
                        val = outw_v[r, sl] / db + bias_v[sl]
                        outw_v[r, sl] = jnp.maximum(val, 0.0)
                pltpu.sync_copy(outw_v, z_ref.at[pl.ds(r0, RD)])
                return 0
            lax.fori_loop(0, HALF // RD, _dchunk, 0)

    @pl.when(c == 0)
    def _():
        _phase_d(z0_hbm, b0_hbm)

    @pl.when(c == 1)
    def _():
        _phase_d(z1_hbm, b1_hbm)


def _layer(xa, xb, src, dst, W, al, ar, b):
    wa = W[:HALF]
    wb = W[HALF:]
    amat = jnp.zeros((F, HALF), jnp.float32)
    amat = amat.at[:, 0].set(al).at[:, 1].set(ar)
    h0, h1, ea = _tc_project(xa, xb, wa, wb, amat)
    el = ea[:, 0].reshape(NR, HALF)
    er = ea[:, 1].reshape(NR, HALF)
    ee, den = _sc_stats(el, er, src, dst)
    z0, z1 = _sc_agg(h0, h1, ee, den, src, dst, b[:HALF], b[HALF:])
    return z0, z1


@jax.jit
def kernel(x, edge_index, W1, al1, ar1, b1, W2, al2, ar2, b2, W3, al3, ar3, b3):
    src = edge_index[0]
    dst = edge_index[1]
    xp = jnp.pad(x, ((0, NP - N), (0, 0)))
    xa = xp[:, :HALF]
    xb = xp[:, HALF:]
    for (W, al, ar, b) in ((W1, al1, ar1, b1), (W2, al2, ar2, b2),
                           (W3, al3, ar3, b3)):
        xa, xb = _layer(xa, xb, src, dst, W, al, ar, b)
    return jnp.concatenate([xa, xb], axis=1)[:N]


# speedup vs baseline: 16.7205x; 1.1841x over previous
import functools

import jax
import jax.numpy as jnp
from jax import lax
from jax.experimental import pallas as pl
from jax.experimental.pallas import tpu as pltpu
from jax.experimental.pallas import tpu_sc as plsc

N = 10000
E = 160000
F = 256
HALF = 128
NT = 16
EPT = E // NT
NV = EPT // 16
NP = 10240
STRIPE = NP // NT
LANES = 16
NR = NP // HALF
KC = 80
NCH = EPT // KC
RD = 16


def _tc_body(xa_ref, xb_ref, wa_ref, wb_ref, a_ref, h0_ref, h1_ref, ea_ref):
    h = jnp.dot(xa_ref[...], wa_ref[...], preferred_element_type=jnp.float32)
    h = h + jnp.dot(xb_ref[...], wb_ref[...], preferred_element_type=jnp.float32)
    h0_ref[...] = h[:, :HALF]
    h1_ref[...] = h[:, HALF:]
    ea_ref[...] = jnp.dot(h, a_ref[...], preferred_element_type=jnp.float32)


def _tc_project(xa, xb, wa, wb, amat):
    mb = 1024
    return pl.pallas_call(
        _tc_body,
        grid=(NP // mb,),
        in_specs=[
            pl.BlockSpec((mb, HALF), lambda i: (i, 0)),
            pl.BlockSpec((mb, HALF), lambda i: (i, 0)),
            pl.BlockSpec((HALF, F), lambda i: (0, 0)),
            pl.BlockSpec((HALF, F), lambda i: (0, 0)),
            pl.BlockSpec((F, HALF), lambda i: (0, 0)),
        ],
        out_specs=[
            pl.BlockSpec((mb, HALF), lambda i: (i, 0)),
            pl.BlockSpec((mb, HALF), lambda i: (i, 0)),
            pl.BlockSpec((mb, HALF), lambda i: (i, 0)),
        ],
        out_shape=[
            jax.ShapeDtypeStruct((NP, HALF), jnp.float32),
            jax.ShapeDtypeStruct((NP, HALF), jnp.float32),
            jax.ShapeDtypeStruct((NP, HALF), jnp.float32),
        ],
    )(xa, xb, wa, wb, amat)


_SC_MESH = plsc.VectorSubcoreMesh(core_axis_name="c", subcore_axis_name="s")


@functools.partial(
    pl.kernel,
    out_type=[
        jax.ShapeDtypeStruct((E,), jnp.float32),
        jax.ShapeDtypeStruct((NR, HALF), jnp.float32),
    ],
    mesh=_SC_MESH,
    compiler_params=pltpu.CompilerParams(needs_layout_passes=False),
    scratch_types=[
        pltpu.VMEM((NR, HALF), jnp.float32),
        pltpu.VMEM((EPT,), jnp.float32),
        pltpu.VMEM((EPT,), jnp.int32),
        pltpu.VMEM((EPT,), jnp.int32),
        pltpu.VMEM((NR,), jnp.int32),
        pltpu.VMEM_SHARED((NR, HALF), jnp.float32),
        pltpu.SemaphoreType.DMA,
    ],
)
def _sc_stats(el_hbm, er_hbm, src_hbm, dst_hbm, ee_hbm, den_hbm,
              work_v, alp_v, src_v, dst_v, idxden, den_sh, sem):
    c = lax.axis_index("c")
    s = lax.axis_index("s")
    ebase = s * EPT
    lane = lax.iota(jnp.int32, LANES)
    c127 = jnp.full((LANES,), 127, jnp.int32)

    pltpu.sync_copy(el_hbm, work_v)
    pltpu.sync_copy(src_hbm.at[pl.ds(ebase, EPT)], src_v)
    pltpu.sync_copy(dst_hbm.at[pl.ds(ebase, EPT)], dst_v)
    for i in range(NR // LANES):
        idxden[pl.ds(i * LANES, LANES)] = lane + (i * LANES)

    def _mx(i, m):
        return jnp.maximum(m, work_v[i >> 3, pl.ds((i & 7) * LANES, LANES)])
    mvec = lax.fori_loop(0, NP // LANES, _mx,
                         jnp.full((LANES,), -jnp.inf, jnp.float32))

    def _lane_gather(v, idx):
        dn = lax.GatherDimensionNumbers(
            offset_dims=(), collapsed_slice_dims=(0,), start_index_map=(0,))
        return lax.gather(v, idx[:, None], dn, (1,),
                          mode=lax.GatherScatterMode.PROMISE_IN_BOUNDS)

    for sh in (1, 2, 4, 8):
        mvec = jnp.maximum(mvec, _lane_gather(mvec, lane ^ sh))
    maxel = mvec

    def _l1a(j, _):
        si = src_v[pl.ds(j * LANES, LANES)]
        g1 = plsc.load_gather(
            work_v, [lax.shift_right_logical(si, 7), si & c127])
        alp_v[pl.ds(j * LANES, LANES)] = g1
        return 0
    lax.fori_loop(0, NV, _l1a, 0)

    pltpu.sync_copy(er_hbm, work_v)

    def _l1b(j, _):
        sl = pl.ds(j * LANES, LANES)
        di = dst_v[sl]
        g2 = plsc.load_gather(
            work_v, [lax.shift_right_logical(di, 7), di & c127])
        t = alp_v[sl] + g2
        e = jnp.where(t >= 0, t, 0.2 * t)
        mt = maxel + g2
        md = jnp.where(mt >= 0, mt, 0.2 * mt)
        alp_v[sl] = jnp.exp(e - md)
        return 0
    lax.fori_loop(0, NV, _l1b, 0)

    def _zw(r, _):
        for v in range(HALF // LANES):
            work_v[r, pl.ds(v * LANES, LANES)] = jnp.zeros((LANES,),
                                                           jnp.float32)
        return 0
    lax.fori_loop(0, NR, _zw, 0)

    @pl.when(s < 10)
    def _():
        pltpu.sync_copy(work_v.at[pl.ds(0, 8)], den_sh.at[pl.ds(s * 8, 8)])
    plsc.subcore_barrier()

    def _l2(j, _):
        sl = pl.ds(j * LANES, LANES)
        di = dst_v[sl]
        plsc.addupdate_scatter(
            work_v, [lax.shift_right_logical(di, 7), di & c127], alp_v[sl])
        return 0
    lax.fori_loop(0, NV, _l2, 0)

    pltpu.async_copy(work_v, den_sh.at[idxden], sem, add=True).wait()
    plsc.subcore_barrier()

    @pl.when(c == 0)
    def _():
        pltpu.sync_copy(alp_v, ee_hbm.at[pl.ds(ebase, EPT)])

        @pl.when(s < 10)
        def _():
            pltpu.sync_copy(den_sh.at[pl.ds(s * 8, 8)],
                            den_hbm.at[pl.ds(s * 8, 8)])


@functools.partial(
    pl.kernel,
    out_type=[
        jax.ShapeDtypeStruct((NP, HALF), jnp.float32),
        jax.ShapeDtypeStruct((NP, HALF), jnp.float32),
    ],
    mesh=_SC_MESH,
    compiler_params=pltpu.CompilerParams(needs_layout_passes=False),
    scratch_types=[
        pltpu.VMEM((3, KC, HALF), jnp.float32),
        pltpu.VMEM((3 * KC,), jnp.int32),
        pltpu.VMEM((3, KC), jnp.int32),
        pltpu.VMEM((3 * KC,), jnp.int32),
        pltpu.VMEM((3 * KC,), jnp.float32),
        pltpu.VMEM((RD, HALF), jnp.float32),
        pltpu.VMEM((HALF,), jnp.float32),
        pltpu.VMEM((HALF,), jnp.float32),
        pltpu.VMEM_SHARED((NP, HALF), jnp.float32),
        pltpu.SemaphoreType.DMA,
        pltpu.SemaphoreType.DMA,
    ],
)
def _sc_agg(h0_hbm, h1_hbm, ee_hbm, den_hbm, src_hbm, dst_hbm, b0_hbm, b1_hbm,
            z0_hbm, z1_hbm,
            rows_v, srcc_v, idx2, dstc_v, eec_v, outw_v, denb_v, bias_v,
            acc_sh, gsem, ssem):
    c = lax.axis_index("c")
    s = lax.axis_index("s")
    ebase = s * EPT

    def _zr(r, _):
        for v in range(HALF // LANES):
            rows_v[0, r, pl.ds(v * LANES, LANES)] = jnp.zeros((LANES,),
                                                              jnp.float32)
        return 0
    lax.fori_loop(0, KC, _zr, 0)

    for q in range(STRIPE // NR):
        pltpu.sync_copy(rows_v.at[0], acc_sh.at[pl.ds(s * STRIPE + q * NR,
                                                      NR)])
    plsc.subcore_barrier()

    def _phase_c(h_ref):
        def _stage_group(ch0, nb):
            eb = ebase + ch0 * KC
            pltpu.sync_copy(src_hbm.at[pl.ds(eb, nb * KC)],
                            srcc_v.at[pl.ds(0, nb * KC)])
            pltpu.sync_copy(dst_hbm.at[pl.ds(eb, nb * KC)],
                            dstc_v.at[pl.ds(0, nb * KC)])
            pltpu.sync_copy(ee_hbm.at[pl.ds(eb, nb * KC)],
                            eec_v.at[pl.ds(0, nb * KC)])
            for b in range(nb):
                for g in range(KC // LANES):
                    idx2[b, pl.ds(g * LANES, LANES)] =                         dstc_v[pl.ds(b * KC + g * LANES, LANES)]

        def _gather(b):
            return pltpu.async_copy(
                h_ref.at[srcc_v.at[pl.ds(b * KC, KC)]], rows_v.at[b], gsem)

        def _scale(b):
            def _sg(g, _2):
                av = eec_v[pl.ds(b * KC + g * LANES, LANES)]
                for k in range(LANES):
                    r = g * LANES + k
                    ab = jnp.broadcast_to(av[k], (LANES,))
                    for v in range(HALF // LANES):
                        sl = pl.ds(v * LANES, LANES)
                        rows_v[b, r, sl] = rows_v[b, r, sl] * ab
                return 0
            lax.fori_loop(0, KC // LANES, _sg, 0)

        def _scatter(b):
            return pltpu.async_copy(rows_v.at[b], acc_sh.at[idx2.at[b]],
                                    ssem, add=True)

        def _run_group(ch0, nb):
            _stage_group(ch0, nb)
            g = [_gather(b) for b in range(nb)]
            d = []
            for b in range(nb):
                g[b].wait()
                _scale(b)
                d.append(_scatter(b))
            for b in range(nb):
                d[b].wait()

        def _group(m, _):
            _run_group(3 * m, 3)
            return 0
        lax.fori_loop(0, NCH // 3, _group, 0)
        if NCH % 3:
            _run_group((NCH // 3) * 3, NCH % 3)

    @pl.when(c == 0)
    def _():
        _phase_c(h0_hbm)

    @pl.when(c == 1)
    def _():
        _phase_c(h1_hbm)

    plsc.subcore_barrier()

    plsc.subcore_barrier()

    def _phase_d(z_ref, b_ref):
        pltpu.sync_copy(b_ref, bias_v)
        for p in range(STRIPE // HALF):
            pltpu.sync_copy(den_hbm.at[s * (STRIPE // HALF) + p], denb_v)

            def _dchunk(q, _):
                r0 = s * STRIPE + p * HALF + q * RD
                pltpu.sync_copy(acc_sh.at[pl.ds(r0, RD)], outw_v)
                for rg in range(RD // LANES):
                    dd = denb_v[pl.ds(q * RD + rg * LANES, LANES)]
                    dd = jnp.where(dd > 0.0, dd, 1.0)
                    for k in range(LANES):
                        r = rg * LANES + k
                        db = jnp.broadcast_to(dd[k], (LANES,))
                        for v in range(HALF // LANES):
                            sl = pl.ds(v * LANES, LANES)
                            val = outw_v[r, sl] / db + bias_v[sl]
                            outw_v[r, sl] = jnp.maximum(val, 0.0)
                pltpu.sync_copy(outw_v, z_ref.at[pl.ds(r0, RD)])
                return 0
            lax.fori_loop(0, HALF // RD, _dchunk, 0)

    @pl.when(c == 0)
    def _():
        _phase_d(z0_hbm, b0_hbm)

    @pl.when(c == 1)
    def _():
        _phase_d(z1_hbm, b1_hbm)


def _layer(xa, xb, src, dst, W, al, ar, b):
    wa = W[:HALF]
    wb = W[HALF:]
    amat = jnp.zeros((F, HALF), jnp.float32)
    amat = amat.at[:, 0].set(al).at[:, 1].set(ar)
    h0, h1, ea = _tc_project(xa, xb, wa, wb, amat)
    el = ea[:, 0].reshape(NR, HALF)
    er = ea[:, 1].reshape(NR, HALF)
    ee, den = _sc_stats(el, er, src, dst)
    z0, z1 = _sc_agg(h0, h1, ee, den, src, dst, b[:HALF], b[HALF:])
    return z0, z1


@jax.jit
def kernel(x, edge_index, W1, al1, ar1, b1, W2, al2, ar2, b2, W3, al3, ar3, b3):
    src = edge_index[0]
    dst = edge_index[1]
    xp = jnp.pad(x, ((0, NP - N), (0, 0)))
    xa = xp[:, :HALF]
    xb = xp[:, HALF:]
    for (W, al, ar, b) in ((W1, al1, ar1, b1), (W2, al2, ar2, b2),
                           (W3, al3, ar3, b3)):
        xa, xb = _layer(xa, xb, src, dst, W, al, ar, b)
    return jnp.concatenate([xa, xb], axis=1)[:N]


# speedup vs baseline: 17.4392x; 1.0430x over previous
import functools

import jax
import jax.numpy as jnp
from jax import lax
from jax.experimental import pallas as pl
from jax.experimental.pallas import tpu as pltpu
from jax.experimental.pallas import tpu_sc as plsc

N = 10000
E = 160000
F = 256
HALF = 128
NT = 16
EPT = E // NT
NV = EPT // 16
NP = 10240
STRIPE = NP // NT
LANES = 16
NR = NP // HALF
KC = 80
NCH = EPT // KC
RD = 16


def _tc_body(xa_ref, xb_ref, wa_ref, wb_ref, a_ref, h0_ref, h1_ref, ea_ref):
    h = jnp.dot(xa_ref[...], wa_ref[...], preferred_element_type=jnp.float32)
    h = h + jnp.dot(xb_ref[...], wb_ref[...], preferred_element_type=jnp.float32)
    h0_ref[...] = h[:, :HALF]
    h1_ref[...] = h[:, HALF:]
    ea_ref[...] = jnp.dot(h, a_ref[...], preferred_element_type=jnp.float32)


def _tc_project(xa, xb, wa, wb, amat):
    mb = 1024
    return pl.pallas_call(
        _tc_body,
        grid=(NP // mb,),
        in_specs=[
            pl.BlockSpec((mb, HALF), lambda i: (i, 0)),
            pl.BlockSpec((mb, HALF), lambda i: (i, 0)),
            pl.BlockSpec((HALF, F), lambda i: (0, 0)),
            pl.BlockSpec((HALF, F), lambda i: (0, 0)),
            pl.BlockSpec((F, HALF), lambda i: (0, 0)),
        ],
        out_specs=[
            pl.BlockSpec((mb, HALF), lambda i: (i, 0)),
            pl.BlockSpec((mb, HALF), lambda i: (i, 0)),
            pl.BlockSpec((mb, HALF), lambda i: (i, 0)),
        ],
        out_shape=[
            jax.ShapeDtypeStruct((NP, HALF), jnp.float32),
            jax.ShapeDtypeStruct((NP, HALF), jnp.float32),
            jax.ShapeDtypeStruct((NP, HALF), jnp.float32),
        ],
    )(xa, xb, wa, wb, amat)


_SC_MESH = plsc.VectorSubcoreMesh(core_axis_name="c", subcore_axis_name="s")


@functools.partial(
    pl.kernel,
    out_type=[
        jax.ShapeDtypeStruct((E,), jnp.float32),
        jax.ShapeDtypeStruct((NR, HALF), jnp.float32),
    ],
    mesh=_SC_MESH,
    compiler_params=pltpu.CompilerParams(needs_layout_passes=False),
    scratch_types=[
        pltpu.VMEM((NR, HALF), jnp.float32),
        pltpu.VMEM((EPT,), jnp.float32),
        pltpu.VMEM((EPT,), jnp.int32),
        pltpu.VMEM((EPT,), jnp.int32),
        pltpu.VMEM((NR,), jnp.int32),
        pltpu.VMEM_SHARED((NR, HALF), jnp.float32),
        pltpu.SemaphoreType.DMA,
    ],
)
def _sc_stats(el_hbm, er_hbm, src_hbm, dst_hbm, ee_hbm, den_hbm,
              work_v, alp_v, src_v, dst_v, idxden, den_sh, sem):
    c = lax.axis_index("c")
    s = lax.axis_index("s")
    ebase = s * EPT
    lane = lax.iota(jnp.int32, LANES)
    c127 = jnp.full((LANES,), 127, jnp.int32)

    pltpu.sync_copy(el_hbm, work_v)
    pltpu.sync_copy(src_hbm.at[pl.ds(ebase, EPT)], src_v)
    pltpu.sync_copy(dst_hbm.at[pl.ds(ebase, EPT)], dst_v)
    for i in range(NR // LANES):
        idxden[pl.ds(i * LANES, LANES)] = lane + (i * LANES)

    def _mx(i, m):
        return jnp.maximum(m, work_v[i >> 3, pl.ds((i & 7) * LANES, LANES)])
    mvec = lax.fori_loop(0, NP // LANES, _mx,
                         jnp.full((LANES,), -jnp.inf, jnp.float32))

    def _lane_gather(v, idx):
        dn = lax.GatherDimensionNumbers(
            offset_dims=(), collapsed_slice_dims=(0,), start_index_map=(0,))
        return lax.gather(v, idx[:, None], dn, (1,),
                          mode=lax.GatherScatterMode.PROMISE_IN_BOUNDS)

    for sh in (1, 2, 4, 8):
        mvec = jnp.maximum(mvec, _lane_gather(mvec, lane ^ sh))
    maxel = mvec

    def _l1a(j, _):
        si = src_v[pl.ds(j * LANES, LANES)]
        g1 = plsc.load_gather(
            work_v, [lax.shift_right_logical(si, 7), si & c127])
        alp_v[pl.ds(j * LANES, LANES)] = g1
        return 0
    lax.fori_loop(0, NV, _l1a, 0)

    pltpu.sync_copy(er_hbm, work_v)

    def _l1b(j, _):
        sl = pl.ds(j * LANES, LANES)
        di = dst_v[sl]
        g2 = plsc.load_gather(
            work_v, [lax.shift_right_logical(di, 7), di & c127])
        t = alp_v[sl] + g2
        e = jnp.where(t >= 0, t, 0.2 * t)
        mt = maxel + g2
        md = jnp.where(mt >= 0, mt, 0.2 * mt)
        alp_v[sl] = jnp.exp(e - md)
        return 0
    lax.fori_loop(0, NV, _l1b, 0)

    def _zw(r, _):
        for v in range(HALF // LANES):
            work_v[r, pl.ds(v * LANES, LANES)] = jnp.zeros((LANES,),
                                                           jnp.float32)
        return 0
    lax.fori_loop(0, NR, _zw, 0)

    @pl.when(s < 10)
    def _():
        pltpu.sync_copy(work_v.at[pl.ds(0, 8)], den_sh.at[pl.ds(s * 8, 8)])
    plsc.subcore_barrier()

    def _l2(j, _):
        sl = pl.ds(j * LANES, LANES)
        di = dst_v[sl]
        plsc.addupdate_scatter(
            work_v, [lax.shift_right_logical(di, 7), di & c127], alp_v[sl])
        return 0
    lax.fori_loop(0, NV, _l2, 0)

    pltpu.async_copy(work_v, den_sh.at[idxden], sem, add=True).wait()
    plsc.subcore_barrier()

    @pl.when(c == 0)
    def _():
        pltpu.sync_copy(alp_v, ee_hbm.at[pl.ds(ebase, EPT)])

        @pl.when(s < 10)
        def _():
            pltpu.sync_copy(den_sh.at[pl.ds(s * 8, 8)],
                            den_hbm.at[pl.ds(s * 8, 8)])


@functools.partial(
    pl.kernel,
    out_type=[
        jax.ShapeDtypeStruct((NP, HALF), jnp.float32),
        jax.ShapeDtypeStruct((NP, HALF), jnp.float32),
    ],
    mesh=_SC_MESH,
    compiler_params=pltpu.CompilerParams(needs_layout_passes=False),
    scratch_types=[
        pltpu.VMEM((3, KC, HALF), jnp.float32),
        pltpu.VMEM((4 * KC,), jnp.int32),
        pltpu.VMEM((4, KC), jnp.int32),
        pltpu.VMEM((4 * KC,), jnp.int32),
        pltpu.VMEM((4 * KC,), jnp.float32),
        pltpu.VMEM((RD, HALF), jnp.float32),
        pltpu.VMEM((HALF,), jnp.float32),
        pltpu.VMEM((HALF,), jnp.float32),
        pltpu.VMEM_SHARED((NP, HALF), jnp.float32),
        pltpu.SemaphoreType.DMA,
        pltpu.SemaphoreType.DMA,
    ],
)
def _sc_agg(h0_hbm, h1_hbm, ee_hbm, den_hbm, src_hbm, dst_hbm, b0_hbm, b1_hbm,
            z0_hbm, z1_hbm,
            rows_v, srcc_v, idx2, dstc_v, eec_v, outw_v, denb_v, bias_v,
            acc_sh, gsem, ssem):
    c = lax.axis_index("c")
    s = lax.axis_index("s")
    ebase = s * EPT

    def _zr(r, _):
        for v in range(HALF // LANES):
            rows_v[0, r, pl.ds(v * LANES, LANES)] = jnp.zeros((LANES,),
                                                              jnp.float32)
        return 0
    lax.fori_loop(0, KC, _zr, 0)

    for q in range(STRIPE // NR):
        pltpu.sync_copy(rows_v.at[0], acc_sh.at[pl.ds(s * STRIPE + q * NR,
                                                      NR)])
    plsc.subcore_barrier()

    def _phase_c(h_ref):
        def _stage_group(ch0, nb):
            eb = ebase + ch0 * KC
            pltpu.sync_copy(src_hbm.at[pl.ds(eb, nb * KC)],
                            srcc_v.at[pl.ds(0, nb * KC)])
            pltpu.sync_copy(dst_hbm.at[pl.ds(eb, nb * KC)],
                            dstc_v.at[pl.ds(0, nb * KC)])
            pltpu.sync_copy(ee_hbm.at[pl.ds(eb, nb * KC)],
                            eec_v.at[pl.ds(0, nb * KC)])
            for b in range(nb):
                for g in range(KC // LANES):
                    idx2[b, pl.ds(g * LANES, LANES)] =                         dstc_v[pl.ds(b * KC + g * LANES, LANES)]

        def _gather_slot(slot, a):
            return pltpu.async_copy(
                h_ref.at[srcc_v.at[pl.ds(a * KC, KC)]], rows_v.at[slot],
                gsem)

        def _scale_slot(slot, a):
            def _sg(g, _2):
                av = eec_v[pl.ds(a * KC + g * LANES, LANES)]
                for k in range(LANES):
                    r = g * LANES + k
                    ab = jnp.broadcast_to(av[k], (LANES,))
                    for v in range(HALF // LANES):
                        sl = pl.ds(v * LANES, LANES)
                        rows_v[slot, r, sl] = rows_v[slot, r, sl] * ab
                return 0
            lax.fori_loop(0, KC // LANES, _sg, 0)

        def _scatter(b):
            return pltpu.async_copy(rows_v.at[b], acc_sh.at[idx2.at[b]],
                                    ssem, add=True)

        def _run_group(ch0, nb):
            _stage_group(ch0, nb)
            g = {}
            d = {}
            for a in range(min(nb, 2)):
                g[a] = _gather_slot(a % 3, a)
            for a in range(nb):
                g.pop(a).wait()
                _scale_slot(a % 3, a)
                d[a] = _scatter(a % 3)
                nxt = a + 2
                if nxt < nb:
                    prev = nxt - 3
                    if prev >= 0:
                        d.pop(prev).wait()
                    g[nxt] = _gather_slot(nxt % 3, nxt)
            for a in sorted(d):
                d[a].wait()

        def _group(m, _):
            _run_group(4 * m, 4)
            return 0
        lax.fori_loop(0, NCH // 4, _group, 0)
        if NCH % 4:
            _run_group((NCH // 4) * 4, NCH % 4)

    @pl.when(c == 0)
    def _():
        _phase_c(h0_hbm)

    @pl.when(c == 1)
    def _():
        _phase_c(h1_hbm)

    plsc.subcore_barrier()

    plsc.subcore_barrier()

    def _phase_d(z_ref, b_ref):
        pltpu.sync_copy(b_ref, bias_v)
        for p in range(STRIPE // HALF):
            pltpu.sync_copy(den_hbm.at[s * (STRIPE // HALF) + p], denb_v)

            def _dchunk(q, _):
                r0 = s * STRIPE + p * HALF + q * RD
                pltpu.sync_copy(acc_sh.at[pl.ds(r0, RD)], outw_v)
                for rg in range(RD // LANES):
                    dd = denb_v[pl.ds(q * RD + rg * LANES, LANES)]
                    dd = jnp.where(dd > 0.0, dd, 1.0)
                    for k in range(LANES):
                        r = rg * LANES + k
                        db = jnp.broadcast_to(dd[k], (LANES,))
                        for v in range(HALF // LANES):
                            sl = pl.ds(v * LANES, LANES)
                            val = outw_v[r, sl] / db + bias_v[sl]
                            outw_v[r, sl] = jnp.maximum(val, 0.0)
                pltpu.sync_copy(outw_v, z_ref.at[pl.ds(r0, RD)])
                return 0
            lax.fori_loop(0, HALF // RD, _dchunk, 0)

    @pl.when(c == 0)
    def _():
        _phase_d(z0_hbm, b0_hbm)

    @pl.when(c == 1)
    def _():
        _phase_d(z1_hbm, b1_hbm)


def _layer(xa, xb, src, dst, W, al, ar, b):
    wa = W[:HALF]
    wb = W[HALF:]
    amat = jnp.zeros((F, HALF), jnp.float32)
    amat = amat.at[:, 0].set(al).at[:, 1].set(ar)
    h0, h1, ea = _tc_project(xa, xb, wa, wb, amat)
    el = ea[:, 0].reshape(NR, HALF)
    er = ea[:, 1].reshape(NR, HALF)
    ee, den = _sc_stats(el, er, src, dst)
    z0, z1 = _sc_agg(h0, h1, ee, den, src, dst, b[:HALF], b[HALF:])
    return z0, z1


@jax.jit
def kernel(x, edge_index, W1, al1, ar1, b1, W2, al2, ar2, b2, W3, al3, ar3, b3):
    src = edge_index[0]
    dst = edge_index[1]
    xp = jnp.pad(x, ((0, NP - N), (0, 0)))
    xa = xp[:, :HALF]
    xb = xp[:, HALF:]
    for (W, al, ar, b) in ((W1, al1, ar1, b1), (W2, al2, ar2, b2),
                           (W3, al3, ar3, b3)):
        xa, xb = _layer(xa, xb, src, dst, W, al, ar, b)
    return jnp.concatenate([xa, xb], axis=1)[:N]


# speedup vs baseline: 17.4895x; 1.0029x over previous
import functools

import jax
import jax.numpy as jnp
from jax import lax
from jax.experimental import pallas as pl
from jax.experimental.pallas import tpu as pltpu
from jax.experimental.pallas import tpu_sc as plsc

N = 10000
E = 160000
F = 256
HALF = 128
NT = 16
EPT = E // NT
NV = EPT // 16
NP = 10240
STRIPE = NP // NT
LANES = 16
NR = NP // HALF
KC = 80
NCH = EPT // KC
RD = 16


def _tc_body(xa_ref, xb_ref, wa_ref, wb_ref, a_ref, h0_ref, h1_ref, ea_ref):
    h = jnp.dot(xa_ref[...], wa_ref[...], preferred_element_type=jnp.float32)
    h = h + jnp.dot(xb_ref[...], wb_ref[...], preferred_element_type=jnp.float32)
    h0_ref[...] = h[:, :HALF]
    h1_ref[...] = h[:, HALF:]
    ea_ref[...] = jnp.dot(h, a_ref[...], preferred_element_type=jnp.float32)


def _tc_project(xa, xb, wa, wb, amat):
    mb = 1024
    return pl.pallas_call(
        _tc_body,
        grid=(NP // mb,),
        in_specs=[
            pl.BlockSpec((mb, HALF), lambda i: (i, 0)),
            pl.BlockSpec((mb, HALF), lambda i: (i, 0)),
            pl.BlockSpec((HALF, F), lambda i: (0, 0)),
            pl.BlockSpec((HALF, F), lambda i: (0, 0)),
            pl.BlockSpec((F, HALF), lambda i: (0, 0)),
        ],
        out_specs=[
            pl.BlockSpec((mb, HALF), lambda i: (i, 0)),
            pl.BlockSpec((mb, HALF), lambda i: (i, 0)),
            pl.BlockSpec((mb, HALF), lambda i: (i, 0)),
        ],
        out_shape=[
            jax.ShapeDtypeStruct((NP, HALF), jnp.float32),
            jax.ShapeDtypeStruct((NP, HALF), jnp.float32),
            jax.ShapeDtypeStruct((NP, HALF), jnp.float32),
        ],
    )(xa, xb, wa, wb, amat)


_SC_MESH = plsc.VectorSubcoreMesh(core_axis_name="c", subcore_axis_name="s")


@functools.partial(
    pl.kernel,
    out_type=[
        jax.ShapeDtypeStruct((E,), jnp.float32),
        jax.ShapeDtypeStruct((NR, HALF), jnp.float32),
    ],
    mesh=_SC_MESH,
    compiler_params=pltpu.CompilerParams(needs_layout_passes=False),
    scratch_types=[
        pltpu.VMEM((NR, HALF), jnp.float32),
        pltpu.VMEM((EPT,), jnp.float32),
        pltpu.VMEM((EPT,), jnp.int32),
        pltpu.VMEM((EPT,), jnp.int32),
        pltpu.VMEM((NR,), jnp.int32),
        pltpu.VMEM_SHARED((NR, HALF), jnp.float32),
        pltpu.SemaphoreType.DMA,
    ],
)
def _sc_stats(el_hbm, er_hbm, src_hbm, dst_hbm, ee_hbm, den_hbm,
              work_v, alp_v, src_v, dst_v, idxden, den_sh, sem):
    c = lax.axis_index("c")
    s = lax.axis_index("s")
    ebase = s * EPT
    lane = lax.iota(jnp.int32, LANES)
    c127 = jnp.full((LANES,), 127, jnp.int32)

    pltpu.sync_copy(el_hbm, work_v)
    pltpu.sync_copy(src_hbm.at[pl.ds(ebase, EPT)], src_v)
    pltpu.sync_copy(dst_hbm.at[pl.ds(ebase, EPT)], dst_v)
    for i in range(NR // LANES):
        idxden[pl.ds(i * LANES, LANES)] = lane + (i * LANES)

    def _mx(i, m):
        return jnp.maximum(m, work_v[i >> 3, pl.ds((i & 7) * LANES, LANES)])
    mvec = lax.fori_loop(0, NP // LANES, _mx,
                         jnp.full((LANES,), -jnp.inf, jnp.float32))

    def _lane_gather(v, idx):
        dn = lax.GatherDimensionNumbers(
            offset_dims=(), collapsed_slice_dims=(0,), start_index_map=(0,))
        return lax.gather(v, idx[:, None], dn, (1,),
                          mode=lax.GatherScatterMode.PROMISE_IN_BOUNDS)

    for sh in (1, 2, 4, 8):
        mvec = jnp.maximum(mvec, _lane_gather(mvec, lane ^ sh))
    maxel = mvec

    def _l1a(j, _):
        si = src_v[pl.ds(j * LANES, LANES)]
        g1 = plsc.load_gather(
            work_v, [lax.shift_right_logical(si, 7), si & c127])
        alp_v[pl.ds(j * LANES, LANES)] = g1
        return 0
    lax.fori_loop(0, NV, _l1a, 0)

    pltpu.sync_copy(er_hbm, work_v)

    def _l1b(j, _):
        sl = pl.ds(j * LANES, LANES)
        di = dst_v[sl]
        g2 = plsc.load_gather(
            work_v, [lax.shift_right_logical(di, 7), di & c127])
        t = alp_v[sl] + g2
        e = jnp.where(t >= 0, t, 0.2 * t)
        mt = maxel + g2
        md = jnp.where(mt >= 0, mt, 0.2 * mt)
        alp_v[sl] = jnp.exp(e - md)
        return 0
    lax.fori_loop(0, NV, _l1b, 0)

    def _zw(r, _):
        for v in range(HALF // LANES):
            work_v[r, pl.ds(v * LANES, LANES)] = jnp.zeros((LANES,),
                                                           jnp.float32)
        return 0
    lax.fori_loop(0, NR, _zw, 0)

    @pl.when(s < 10)
    def _():
        pltpu.sync_copy(work_v.at[pl.ds(0, 8)], den_sh.at[pl.ds(s * 8, 8)])
    plsc.subcore_barrier()

    def _l2(j, _):
        sl = pl.ds(j * LANES, LANES)
        di = dst_v[sl]
        plsc.addupdate_scatter(
            work_v, [lax.shift_right_logical(di, 7), di & c127], alp_v[sl])
        return 0
    lax.fori_loop(0, NV, _l2, 0)

    pltpu.async_copy(work_v, den_sh.at[idxden], sem, add=True).wait()
    plsc.subcore_barrier()

    @pl.when(c == 0)
    def _():
        pltpu.sync_copy(alp_v, ee_hbm.at[pl.ds(ebase, EPT)])

        @pl.when(s < 10)
        def _():
            pltpu.sync_copy(den_sh.at[pl.ds(s * 8, 8)],
                            den_hbm.at[pl.ds(s * 8, 8)])


@functools.partial(
    pl.kernel,
    out_type=[
        jax.ShapeDtypeStruct((NP, HALF), jnp.float32),
        jax.ShapeDtypeStruct((NP, HALF), jnp.float32),
    ],
    mesh=_SC_MESH,
    compiler_params=pltpu.CompilerParams(needs_layout_passes=False),
    scratch_types=[
        pltpu.VMEM((3, KC, HALF), jnp.float32),
        pltpu.VMEM((4 * KC,), jnp.int32),
        pltpu.VMEM((4, KC), jnp.int32),
        pltpu.VMEM((4 * KC,), jnp.int32),
        pltpu.VMEM((4 * KC,), jnp.float32),
        pltpu.VMEM((RD, HALF), jnp.float32),
        pltpu.VMEM((HALF,), jnp.float32),
        pltpu.VMEM((HALF,), jnp.float32),
        pltpu.VMEM_SHARED((NP, HALF), jnp.float32),
        pltpu.SemaphoreType.DMA,
        pltpu.SemaphoreType.DMA,
    ],
)
def _sc_agg(h0_hbm, h1_hbm, ee_hbm, den_hbm, src_hbm, dst_hbm, b0_hbm, b1_hbm,
            z0_hbm, z1_hbm,
            rows_v, srcc_v, idx2, dstc_v, eec_v, outw_v, denb_v, bias_v,
            acc_sh, gsem, ssem):
    c = lax.axis_index("c")
    s = lax.axis_index("s")
    ebase = s * EPT

    def _zr(r, _):
        for v in range(HALF // LANES):
            rows_v[0, r, pl.ds(v * LANES, LANES)] = jnp.zeros((LANES,),
                                                              jnp.float32)
        return 0
    lax.fori_loop(0, KC, _zr, 0)

    for q in range(STRIPE // NR):
        pltpu.sync_copy(rows_v.at[0], acc_sh.at[pl.ds(s * STRIPE + q * NR,
                                                      NR)])
    plsc.subcore_barrier()

    def _phase_c(h_ref):
        def _stage_group(ch0, nb):
            eb = ebase + ch0 * KC
            pltpu.sync_copy(src_hbm.at[pl.ds(eb, nb * KC)],
                            srcc_v.at[pl.ds(0, nb * KC)])
            pltpu.sync_copy(dst_hbm.at[pl.ds(eb, nb * KC)],
                            dstc_v.at[pl.ds(0, nb * KC)])
            pltpu.sync_copy(ee_hbm.at[pl.ds(eb, nb * KC)],
                            eec_v.at[pl.ds(0, nb * KC)])
            for b in range(nb):
                for g in range(KC // LANES):
                    idx2[b, pl.ds(g * LANES, LANES)] =                         dstc_v[pl.ds(b * KC + g * LANES, LANES)]

        def _gather_slot(slot, a):
            return pltpu.async_copy(
                h_ref.at[srcc_v.at[pl.ds(a * KC, KC)]], rows_v.at[slot],
                gsem)

        def _scale_slot(slot, a):
            def _sg(g, _2):
                av = eec_v[pl.ds(a * KC + g * LANES, LANES)]
                for k in range(LANES):
                    r = g * LANES + k
                    ab = jnp.broadcast_to(av[k], (LANES,))
                    for v in range(HALF // LANES):
                        sl = pl.ds(v * LANES, LANES)
                        rows_v[slot, r, sl] = rows_v[slot, r, sl] * ab
                return 0
            lax.fori_loop(0, KC // LANES, _sg, 0)

        def _scatter(slot, a):
            return pltpu.async_copy(rows_v.at[slot], acc_sh.at[idx2.at[a]],
                                    ssem, add=True)

        def _run_group(ch0, nb):
            _stage_group(ch0, nb)
            g = {}
            d = {}
            for a in range(min(nb, 2)):
                g[a] = _gather_slot(a % 3, a)
            for a in range(nb):
                g.pop(a).wait()
                _scale_slot(a % 3, a)
                d[a] = _scatter(a % 3, a)
                nxt = a + 2
                if nxt < nb:
                    prev = nxt - 3
                    if prev >= 0:
                        d.pop(prev).wait()
                    g[nxt] = _gather_slot(nxt % 3, nxt)
            for a in sorted(d):
                d[a].wait()

        def _group(m, _):
            _run_group(4 * m, 4)
            return 0
        lax.fori_loop(0, NCH // 4, _group, 0)
        if NCH % 4:
            _run_group((NCH // 4) * 4, NCH % 4)

    @pl.when(c == 0)
    def _():
        _phase_c(h0_hbm)

    @pl.when(c == 1)
    def _():
        _phase_c(h1_hbm)

    plsc.subcore_barrier()

    plsc.subcore_barrier()

    def _phase_d(z_ref, b_ref):
        pltpu.sync_copy(b_ref, bias_v)
        for p in range(STRIPE // HALF):
            pltpu.sync_copy(den_hbm.at[s * (STRIPE // HALF) + p], denb_v)

            def _dchunk(q, _):
                r0 = s * STRIPE + p * HALF + q * RD
                pltpu.sync_copy(acc_sh.at[pl.ds(r0, RD)], outw_v)
                for rg in range(RD // LANES):
                    dd = denb_v[pl.ds(q * RD + rg * LANES, LANES)]
                    dd = jnp.where(dd > 0.0, dd, 1.0)
                    for k in range(LANES):
                        r = rg * LANES + k
                        db = jnp.broadcast_to(dd[k], (LANES,))
                        for v in range(HALF // LANES):
                            sl = pl.ds(v * LANES, LANES)
                            val = outw_v[r, sl] / db + bias_v[sl]
                            outw_v[r, sl] = jnp.maximum(val, 0.0)
                pltpu.sync_copy(outw_v, z_ref.at[pl.ds(r0, RD)])
                return 0
            lax.fori_loop(0, HALF // RD, _dchunk, 0)

    @pl.when(c == 0)
    def _():
        _phase_d(z0_hbm, b0_hbm)

    @pl.when(c == 1)
    def _():
        _phase_d(z1_hbm, b1_hbm)


def _layer(xa, xb, src, dst, W, al, ar, b):
    wa = W[:HALF]
    wb = W[HALF:]
    amat = jnp.zeros((F, HALF), jnp.float32)
    amat = amat.at[:, 0].set(al).at[:, 1].set(ar)
    h0, h1, ea = _tc_project(xa, xb, wa, wb, amat)
    el = ea[:, 0].reshape(NR, HALF)
    er = ea[:, 1].reshape(NR, HALF)
    ee, den = _sc_stats(el, er, src, dst)
    z0, z1 = _sc_agg(h0, h1, ee, den, src, dst, b[:HALF], b[HALF:])
    return z0, z1


@jax.jit
def kernel(x, edge_index, W1, al1, ar1, b1, W2, al2, ar2, b2, W3, al3, ar3, b3):
    src = edge_index[0]
    dst = edge_index[1]
    xp = jnp.pad(x, ((0, NP - N), (0, 0)))
    xa = xp[:, :HALF]
    xb = xp[:, HALF:]
    for (W, al, ar, b) in ((W1, al1, ar1, b1), (W2, al2, ar2, b2),
                           (W3, al3, ar3, b3)):
        xa, xb = _layer(xa, xb, src, dst, W, al, ar, b)
    return jnp.concatenate([xa, xb], axis=1)[:N]
